# baseline (speedup 1.0000x reference)
import jax
import jax.numpy as jnp
from jax.experimental import pallas as pl

N = 100
E = 4000
K = 50
F1 = 64
F2 = 20
NEG = 0.01

_HI = jax.lax.Precision.HIGHEST
_DEF = jax.lax.Precision.DEFAULT


def _dot(a, b, prec=_HI):
    return jax.lax.dot_general(a, b, (((1,), (0,)), ((), ())),
                               preferred_element_type=jnp.float32,
                               precision=prec)


def _dotd(a, b):
    return _dot(a, b, _DEF)


def _dotT0(a, b, prec=_HI):
    return jax.lax.dot_general(a, b, (((0,), (0,)), ((), ())),
                               preferred_element_type=jnp.float32,
                               precision=prec)


def _dotT1(a, b, prec=_HI):
    return jax.lax.dot_general(a, b, (((1,), (1,)), ((), ())),
                               preferred_element_type=jnp.float32,
                               precision=prec)


def _smax(z):
    m = jnp.max(z, axis=1, keepdims=True)
    e = jnp.exp(z - m)
    return e / jnp.sum(e, axis=1, keepdims=True)


def _leaky(z):
    return jnp.where(z >= 0, z, NEG * z)


def _mega(ei_ref, wT_ref, x_ref,
          wl1_ref, bl1_ref, wr1_ref, br1_ref,
          wl2_ref, bl2_ref, wr2_ref, br2_ref,
          wg1_ref, bg1_ref, wrel_ref, brel_ref, wroot_ref,
          wc0_ref, wc1_ref, wc2_ref, bc_ref,
          out_ref):
    f32 = jnp.float32
    i32 = jnp.int32

    iN_r = jax.lax.broadcasted_iota(i32, (N, N), 0)
    iN_c = jax.lax.broadcasted_iota(i32, (N, N), 1)
    eyeN = (iN_r == iN_c).astype(f32)

    rowT = ei_ref[0:1, :]
    colT = ei_ref[1:2, :]
    wT = wT_ref[...]

    iotaNE = jax.lax.broadcasted_iota(i32, (N, E), 0)
    eqc = iotaNE == colT
    ohcT = jnp.where(eqc, 1.0, 0.0).astype(f32)
    ohrT = (iotaNE == rowT).astype(f32)
    bf16 = jnp.bfloat16
    w1 = wT.astype(bf16).astype(f32)
    r1 = wT - w1
    w2 = r1.astype(bf16).astype(f32)
    w3 = (r1 - w2).astype(bf16).astype(f32)
    A_g = (_dotT1(jnp.where(eqc, w1, 0.0), ohrT, _DEF)
           + _dotT1(jnp.where(eqc, w2, 0.0), ohrT, _DEF)
           + _dotT1(jnp.where(eqc, w3, 0.0), ohrT, _DEF))
    B = _dotT1(ohcT, ohrT, _DEF)

    maskL = ((iN_r < K) & (iN_c < K)).astype(f32)
    maskR = ((iN_r >= K) & (iN_c >= K)).astype(f32)

    def norm_mat(A):
        deg = jnp.sum(A, axis=1, keepdims=True) + 1.0
        dis = jax.lax.rsqrt(deg)
        disT = _dotT0(dis, eyeN)
        return dis * A * disT + eyeN * (1.0 / deg)

    M_l = norm_mat(A_g * maskL)
    M_r = norm_mat(A_g * maskR)
    M_g = norm_mat(A_g)

    x = x_ref[...]
    rowmask = jax.lax.broadcasted_iota(i32, (N, 1), 0) < K

    hl = _leaky(_dot(M_l, _dotd(x, wl1_ref[...])) + bl1_ref[...])
    hr = _leaky(_dot(M_r, _dotd(x, wr1_ref[...])) + br1_ref[...])
    h1 = jnp.where(rowmask, hl, hr)

    hl2 = _leaky(_dot(M_l, _dotd(h1, wl2_ref[...])) + bl2_ref[...])
    hr2 = _leaky(_dot(M_r, _dotd(h1, wr2_ref[...])) + br2_ref[...])
    h2c = jnp.where(rowmask, hl2, hr2)

    h2 = _leaky(_dot(M_g, _dotd(h2c, wg1_ref[...])) + bg1_ref[...])

    agg = _dot(B, h2)
    score = jnp.tanh(_dotd(agg, wrel_ref[...]) + brel_ref[...]
                     + _dotd(h2, wroot_ref[...]))

    scoreT = _dotT0(score, eyeN)

    beats = (scoreT > score) | ((scoreT == score) & (iN_c < iN_r))
    rank = jnp.sum(beats.astype(f32), axis=1, keepdims=True)
    rank_i = rank.astype(i32)

    PT = (rank_i == jax.lax.broadcasted_iota(i32, (N, K), 1)).astype(f32)

    vals = _dotT0(PT, score)
    x_pool = _dotT0(PT, h2) * vals

    B_pool = _dot(_dotT0(PT, B), PT)
    degch = jnp.sum(B_pool, axis=1, keepdims=True)
    pos_deg = degch > 0
    dch = jnp.where(pos_deg, jax.lax.rsqrt(jnp.where(pos_deg, degch, 1.0)), 0.0)
    iK_r = jax.lax.broadcasted_iota(i32, (K, K), 0)
    iK_c = jax.lax.broadcasted_iota(i32, (K, K), 1)
    eyeK = (iK_r == iK_c).astype(f32)
    dchT = _dotT0(dch, eyeK)
    M_ch = -(dch * B_pool * dchT)

    Etop = (jax.lax.broadcasted_iota(i32, (N, K), 0)
            == jax.lax.broadcasted_iota(i32, (N, K), 1)).astype(f32)
    h2top = h2[0:K, :]
    tx1t = _dot(M_ch, h2top)
    Tx1 = _dot(Etop, tx1t)
    Tx2 = 2.0 * _dot(Etop, _dot(M_ch, tx1t)) - h2
    cheb = (_dotd(h2, wc0_ref[...]) + _dotd(Tx1, wc1_ref[...])
            + _dotd(Tx2, wc2_ref[...]) + bc_ref[...])

    ass = _smax(cheb)
    s = _smax(ass)

    H_coarse = _dotT0(s, h2, _DEF)

    kept = rank_i < K
    le = (iN_c <= iN_r).astype(f32)
    pos = _dot(le, kept.astype(f32))
    pos0 = pos.astype(i32) - 1
    PsortT = ((pos0 == jax.lax.broadcasted_iota(i32, (N, K), 1))
              & kept).astype(f32)
    inter = _dotT0(PsortT, ass)

    out_ref[...] = x_pool + _dotd(inter, H_coarse)


def kernel(x, edge_index, edge_attr, adj, W_l1, b_l1, W_r1, b_r1, W_l2, b_l2,
           W_r2, b_r2, W_g1, b_g1, W_rel, b_rel, W_root, W_c0, W_c1, W_c2, b_c):
    del adj
    wT = edge_attr.reshape(1, E)
    out = pl.pallas_call(
        _mega,
        out_shape=jax.ShapeDtypeStruct((K, F2), jnp.float32),
    )(edge_index, wT, x,
      W_l1, b_l1.reshape(1, F1), W_r1, b_r1.reshape(1, F1),
      W_l2, b_l2.reshape(1, F2), W_r2, b_r2.reshape(1, F2),
      W_g1, b_g1.reshape(1, F2), W_rel, b_rel.reshape(1, 1), W_root,
      W_c0, W_c1, W_c2, b_c.reshape(1, K))
    return out.reshape(1, -1)


# speedup vs baseline: 1.0109x; 1.0109x over previous
import jax
import jax.numpy as jnp
from jax.experimental import pallas as pl

N = 100
E = 4000
K = 50
F1 = 64
F2 = 20
NEG = 0.01

_HI = jax.lax.Precision.HIGHEST
_DEF = jax.lax.Precision.DEFAULT


def _dot(a, b, prec=_HI):
    return jax.lax.dot_general(a, b, (((1,), (0,)), ((), ())),
                               preferred_element_type=jnp.float32,
                               precision=prec)


def _dotd(a, b):
    return _dot(a, b, _DEF)


def _dotT0(a, b, prec=_HI):
    return jax.lax.dot_general(a, b, (((0,), (0,)), ((), ())),
                               preferred_element_type=jnp.float32,
                               precision=prec)


def _dotT1(a, b, prec=_HI):
    return jax.lax.dot_general(a, b, (((1,), (1,)), ((), ())),
                               preferred_element_type=jnp.float32,
                               precision=prec)


def _smax(z):
    m = jnp.max(z, axis=1, keepdims=True)
    e = jnp.exp(z - m)
    return e / jnp.sum(e, axis=1, keepdims=True)


def _leaky(z):
    return jnp.where(z >= 0, z, NEG * z)


def _mega(ei_ref, wT_ref, x_ref,
          wl1_ref, bl1_ref, wr1_ref, br1_ref,
          wl2_ref, bl2_ref, wr2_ref, br2_ref,
          wg1_ref, bg1_ref, wrel_ref, brel_ref, wroot_ref,
          wc0_ref, wc1_ref, wc2_ref, bc_ref,
          out_ref):
    f32 = jnp.float32
    i32 = jnp.int32

    iN_r = jax.lax.broadcasted_iota(i32, (N, N), 0)
    iN_c = jax.lax.broadcasted_iota(i32, (N, N), 1)
    eyeN = (iN_r == iN_c).astype(f32)

    rowT = ei_ref[0:1, :]
    colT = ei_ref[1:2, :]
    wT = wT_ref[...]

    iotaNE = jax.lax.broadcasted_iota(i32, (N, E), 0)
    eqc = iotaNE == colT
    ohcT = jnp.where(eqc, 1.0, 0.0).astype(f32)
    ohrT = (iotaNE == rowT).astype(f32)
    bf16 = jnp.bfloat16
    w1 = wT.astype(bf16).astype(f32)
    r1 = wT - w1
    w2 = r1.astype(bf16).astype(f32)
    w3 = (r1 - w2).astype(bf16).astype(f32)
    A_g = (_dotT1(jnp.where(eqc, w1, 0.0), ohrT, _DEF)
           + _dotT1(jnp.where(eqc, w2, 0.0), ohrT, _DEF)
           + _dotT1(jnp.where(eqc, w3, 0.0), ohrT, _DEF))
    B = _dotT1(ohcT, ohrT, _DEF)

    maskL = ((iN_r < K) & (iN_c < K)).astype(f32)
    maskR = ((iN_r >= K) & (iN_c >= K)).astype(f32)

    def norm_mat(A):
        deg = jnp.sum(A, axis=1, keepdims=True) + 1.0
        dis = jax.lax.rsqrt(deg)
        return _dot(dis * A, eyeN * dis) + eyeN * (1.0 / deg)

    M_l = norm_mat(A_g * maskL)
    M_r = norm_mat(A_g * maskR)
    M_g = norm_mat(A_g)

    x = x_ref[...]
    rowmask = jax.lax.broadcasted_iota(i32, (N, 1), 0) < K

    hl = _leaky(_dot(M_l, _dotd(x, wl1_ref[...])) + bl1_ref[...])
    hr = _leaky(_dot(M_r, _dotd(x, wr1_ref[...])) + br1_ref[...])
    h1 = jnp.where(rowmask, hl, hr)

    hl2 = _leaky(_dot(M_l, _dotd(h1, wl2_ref[...])) + bl2_ref[...])
    hr2 = _leaky(_dot(M_r, _dotd(h1, wr2_ref[...])) + br2_ref[...])
    h2c = jnp.where(rowmask, hl2, hr2)

    h2 = _leaky(_dot(M_g, _dotd(h2c, wg1_ref[...])) + bg1_ref[...])

    agg = _dot(B, h2)
    score = jnp.tanh(_dotd(agg, wrel_ref[...]) + brel_ref[...]
                     + _dotd(h2, wroot_ref[...]))

    ones1N = jnp.ones((1, N), f32)
    scoreT = _dot(ones1N, eyeN * score)

    beats = (scoreT > score) | ((scoreT == score) & (iN_c < iN_r))
    rank = jnp.sum(beats.astype(f32), axis=1, keepdims=True)
    rankT = _dot(ones1N, eyeN * rank)
    rank_i = rank.astype(i32)
    rankT_i = rankT.astype(i32)

    P = (rankT_i == jax.lax.broadcasted_iota(i32, (K, N), 0)).astype(f32)
    PT = (rank_i == jax.lax.broadcasted_iota(i32, (N, K), 1)).astype(f32)

    vals = _dot(P, score)
    x_pool = _dot(P, h2) * vals

    B_pool = _dot(_dot(P, B), PT)
    degch = jnp.sum(B_pool, axis=1, keepdims=True)
    pos_deg = degch > 0
    dch = jnp.where(pos_deg, jax.lax.rsqrt(jnp.where(pos_deg, degch, 1.0)), 0.0)
    iK_r = jax.lax.broadcasted_iota(i32, (K, K), 0)
    iK_c = jax.lax.broadcasted_iota(i32, (K, K), 1)
    eyeK = (iK_r == iK_c).astype(f32)
    M_ch = -_dot(dch * B_pool, eyeK * dch)

    topKN = (jax.lax.broadcasted_iota(i32, (K, N), 0)
             == jax.lax.broadcasted_iota(i32, (K, N), 1)).astype(f32)
    Etop = (jax.lax.broadcasted_iota(i32, (N, K), 0)
            == jax.lax.broadcasted_iota(i32, (N, K), 1)).astype(f32)
    h2top = _dot(topKN, h2)
    tx1t = _dot(M_ch, h2top)
    Tx1 = _dot(Etop, tx1t)
    Tx2 = 2.0 * _dot(Etop, _dot(M_ch, tx1t)) - h2
    cheb = (_dotd(h2, wc0_ref[...]) + _dotd(Tx1, wc1_ref[...])
            + _dotd(Tx2, wc2_ref[...]) + bc_ref[...])

    ass = _smax(cheb)
    s = _smax(ass)

    H_coarse = _dotT0(s, h2, _DEF)

    keptT = (rankT_i < K).astype(f32)
    le = (iN_r <= iN_c).astype(f32)
    posT = _dot(keptT, le)
    pos0T = posT.astype(i32) - 1
    Psort = ((pos0T == jax.lax.broadcasted_iota(i32, (K, N), 0))
             & (rankT_i < K)).astype(f32)
    inter = _dot(Psort, ass)

    out_ref[...] = x_pool + _dotd(inter, H_coarse)


def kernel(x, edge_index, edge_attr, adj, W_l1, b_l1, W_r1, b_r1, W_l2, b_l2,
           W_r2, b_r2, W_g1, b_g1, W_rel, b_rel, W_root, W_c0, W_c1, W_c2, b_c):
    del adj
    wT = edge_attr.reshape(1, E)
    out = pl.pallas_call(
        _mega,
        out_shape=jax.ShapeDtypeStruct((K, F2), jnp.float32),
    )(edge_index, wT, x,
      W_l1, b_l1.reshape(1, F1), W_r1, b_r1.reshape(1, F1),
      W_l2, b_l2.reshape(1, F2), W_r2, b_r2.reshape(1, F2),
      W_g1, b_g1.reshape(1, F2), W_rel, b_rel.reshape(1, 1), W_root,
      W_c0, W_c1, W_c2, b_c.reshape(1, K))
    return out.reshape(1, -1)
